# baseline (speedup 1.0000x reference)
import functools

import jax
import jax.numpy as jnp
from jax import lax
from jax.experimental import pallas as pl
from jax.experimental.pallas import tpu as pltpu


def _round_up(x, m):
    return ((x + m - 1) // m) * m


def _body(feat_hbm, alpha_hbm, wqT_ref, bq_ref, w1T_ref, w2T_ref, bw_ref,
          out_hbm, feat_ref, abuf, h_ref, obuf, sems, osems, fsem,
          *, tiles, tk, depth):
    nk = len(tiles)

    def alpha_copy(i):
        row0, rows = tiles[i]
        slot = i % depth
        return pltpu.make_async_copy(
            alpha_hbm.at[pl.ds(row0, rows), :],
            abuf.at[pl.ds(slot * tk, rows), :],
            sems.at[slot])

    def out_copy(i):
        row0, rows = tiles[i]
        oslot = i % 2
        return pltpu.make_async_copy(
            obuf.at[pl.ds(oslot * tk, rows), :],
            out_hbm.at[pl.ds(row0, rows), :],
            osems.at[oslot])

    fcp = pltpu.make_async_copy(feat_hbm, feat_ref, fsem)
    fcp.start()
    for i in range(min(depth - 1, nk)):
        alpha_copy(i).start()
    fcp.wait()

    x = feat_ref[...].astype(jnp.bfloat16)
    acc = jnp.dot(x, wqT_ref[...], preferred_element_type=jnp.float32)
    h_ref[...] = jnp.maximum(acc + bq_ref[...], 0.0).astype(jnp.bfloat16)

    for i in range(nk):
        row0, rows = tiles[i]
        slot = i % depth
        alpha_copy(i).wait()

        if i + depth - 1 < nk:
            alpha_copy(i + depth - 1).start()

        a16 = abuf[pl.ds(slot * tk, rows), :].astype(jnp.bfloat16)
        hn = jnp.dot(a16, h_ref[...], preferred_element_type=jnp.float32)

        hd = h_ref[pl.ds(row0, rows), :]
        z = jnp.dot(hd, w1T_ref[...], preferred_element_type=jnp.float32)
        z = z + jnp.dot(hn.astype(jnp.bfloat16), w2T_ref[...],
                        preferred_element_type=jnp.float32)
        z = jnp.maximum(z + bw_ref[...], 0.0)
        sumsq = jnp.sum(z * z, axis=-1, keepdims=True)
        inv_norm = lax.rsqrt(sumsq + 1e-12)

        if i >= 2:
            out_copy(i - 2).wait()

        obuf[pl.ds((i % 2) * tk, rows), :] = (z * inv_norm).astype(obuf.dtype)
        out_copy(i).start()

    for i in range(max(0, nk - 2), nk):
        out_copy(i).wait()


def kernel(features, alpha, wq, bq, ww, bw):
    n, in_dim = features.shape
    out_dim = ww.shape[0]
    dtype = features.dtype

    d_pad = _round_up(in_dim, 128)
    o_pad = _round_up(out_dim, 128)
    n_pad = _round_up(n, 128)

    def pad2(x, r, c):
        if x.shape == (r, c):
            return x
        return jnp.pad(x, ((0, r - x.shape[0]), (0, c - x.shape[1])))

    feat_p = pad2(features, n_pad, d_pad)
    alpha_p = pad2(alpha, n_pad, n_pad)
    wqT_p = pad2(wq.T, d_pad, d_pad).astype(jnp.bfloat16)
    bq_p = pad2(bq.reshape(1, in_dim), 1, d_pad)
    w1T_p = pad2(ww[:, :in_dim].T, d_pad, o_pad).astype(jnp.bfloat16)
    w2T_p = pad2(ww[:, in_dim:].T, d_pad, o_pad).astype(jnp.bfloat16)
    bw_p = pad2(bw.reshape(1, out_dim), 1, o_pad)

    if n_pad % 512 == 0 and n_pad >= 1024:
        tk = 512
        sizes = [256, 256] + [512] * ((n_pad - 1024) // 512) + [256, 256]
    else:
        tk = 128
        sizes = [128] * (n_pad // 128)
    tiles, r = [], 0
    for s in sizes:
        tiles.append((r, s))
        r += s
    depth = min(4, len(tiles) + 1)

    out_p = pl.pallas_call(
        functools.partial(_body, tiles=tuple(tiles), tk=tk, depth=depth),
        out_shape=jax.ShapeDtypeStruct((n_pad, o_pad), dtype),
        in_specs=[
            pl.BlockSpec(memory_space=pl.ANY),
            pl.BlockSpec(memory_space=pl.ANY),
            pl.BlockSpec((d_pad, d_pad), lambda: (0, 0)),
            pl.BlockSpec((1, d_pad), lambda: (0, 0)),
            pl.BlockSpec((d_pad, o_pad), lambda: (0, 0)),
            pl.BlockSpec((d_pad, o_pad), lambda: (0, 0)),
            pl.BlockSpec((1, o_pad), lambda: (0, 0)),
        ],
        out_specs=pl.BlockSpec(memory_space=pl.ANY),
        scratch_shapes=[
            pltpu.VMEM((n_pad, d_pad), jnp.float32),
            pltpu.VMEM((depth * tk, n_pad), jnp.float32),
            pltpu.VMEM((n_pad, d_pad), jnp.bfloat16),
            pltpu.VMEM((2 * tk, o_pad), jnp.float32),
            pltpu.SemaphoreType.DMA((depth,)),
            pltpu.SemaphoreType.DMA((2,)),
            pltpu.SemaphoreType.DMA,
        ],
        compiler_params=pltpu.CompilerParams(
            vmem_limit_bytes=58 * 1024 * 1024),
    )(feat_p, alpha_p, wqT_p, bq_p, w1T_p, w2T_p, bw_p)

    return out_p[:n, :out_dim]


# speedup vs baseline: 1.1136x; 1.1136x over previous
import functools

import jax
import jax.numpy as jnp
from jax import lax
from jax.experimental import pallas as pl
from jax.experimental.pallas import tpu as pltpu


def _round_up(x, m):
    return ((x + m - 1) // m) * m


def _body(feat_hbm, alpha_hbm, wqT_ref, bq_ref, w1T_ref, w2T_ref, bw_ref,
          out_hbm, feat_ref, abuf, h_ref, obuf, sems, osems, fsem,
          *, nk, tk, depth):
    def alpha_copy(k, slot):
        return pltpu.make_async_copy(
            alpha_hbm.at[pl.ds(k * tk, tk), :],
            abuf.at[pl.ds(slot * tk, tk), :],
            sems.at[slot])

    def out_copy(k, oslot):
        return pltpu.make_async_copy(
            obuf.at[pl.ds(oslot * tk, tk), :],
            out_hbm.at[pl.ds(k * tk, tk), :],
            osems.at[oslot])

    fcp = pltpu.make_async_copy(feat_hbm, feat_ref, fsem)
    fcp.start()
    for s in range(depth - 1):
        alpha_copy(s, s).start()
    fcp.wait()

    x = feat_ref[...].astype(jnp.bfloat16)
    acc = jnp.dot(x, wqT_ref[...], preferred_element_type=jnp.float32)
    h_ref[...] = jnp.maximum(acc + bq_ref[...], 0.0).astype(jnp.bfloat16)

    def step(k, carry):
        slot = lax.rem(k, depth)
        oslot = lax.rem(k, 2)
        alpha_copy(k, slot).wait()

        @pl.when(k + depth - 1 < nk)
        def _():
            alpha_copy(k + depth - 1, lax.rem(k + depth - 1, depth)).start()

        a16 = abuf[pl.ds(slot * tk, tk), :].astype(jnp.bfloat16)
        hn = jnp.dot(a16, h_ref[...], preferred_element_type=jnp.float32)

        hd = h_ref[pl.ds(k * tk, tk), :]
        z = jnp.dot(hd, w1T_ref[...], preferred_element_type=jnp.float32)
        z = z + jnp.dot(hn.astype(jnp.bfloat16), w2T_ref[...],
                        preferred_element_type=jnp.float32)
        z = jnp.maximum(z + bw_ref[...], 0.0)
        sumsq = jnp.sum(z * z, axis=-1, keepdims=True)
        inv_norm = lax.rsqrt(sumsq + 1e-12)

        @pl.when(k >= 2)
        def _():
            out_copy(k - 2, oslot).wait()

        obuf[pl.ds(oslot * tk, tk), :] = (z * inv_norm).astype(obuf.dtype)
        out_copy(k, oslot).start()
        return carry

    lax.fori_loop(0, nk, step, 0)
    for t in range(min(2, nk)):
        k = nk - min(2, nk) + t
        out_copy(k, k % 2).wait()


def kernel(features, alpha, wq, bq, ww, bw):
    n, in_dim = features.shape
    out_dim = ww.shape[0]
    dtype = features.dtype

    d_pad = _round_up(in_dim, 128)
    o_pad = _round_up(out_dim, 128)
    n_pad = _round_up(n, 128)

    def pad2(x, r, c):
        if x.shape == (r, c):
            return x
        return jnp.pad(x, ((0, r - x.shape[0]), (0, c - x.shape[1])))

    feat_p = pad2(features, n_pad, d_pad)
    alpha_p = pad2(alpha, n_pad, n_pad)
    wqT_p = pad2(wq.T, d_pad, d_pad).astype(jnp.bfloat16)
    bq_p = pad2(bq.reshape(1, in_dim), 1, d_pad)
    w1T_p = pad2(ww[:, :in_dim].T, d_pad, o_pad).astype(jnp.bfloat16)
    w2T_p = pad2(ww[:, in_dim:].T, d_pad, o_pad).astype(jnp.bfloat16)
    bw_p = pad2(bw.reshape(1, out_dim), 1, o_pad)

    tk = 512 if n_pad % 512 == 0 else 128
    nk = n_pad // tk
    depth = min(4, nk + 1)

    out_p = pl.pallas_call(
        functools.partial(_body, nk=nk, tk=tk, depth=depth),
        out_shape=jax.ShapeDtypeStruct((n_pad, o_pad), dtype),
        in_specs=[
            pl.BlockSpec(memory_space=pl.ANY),
            pl.BlockSpec(memory_space=pl.ANY),
            pl.BlockSpec((d_pad, d_pad), lambda: (0, 0)),
            pl.BlockSpec((1, d_pad), lambda: (0, 0)),
            pl.BlockSpec((d_pad, o_pad), lambda: (0, 0)),
            pl.BlockSpec((d_pad, o_pad), lambda: (0, 0)),
            pl.BlockSpec((1, o_pad), lambda: (0, 0)),
        ],
        out_specs=pl.BlockSpec(memory_space=pl.ANY),
        scratch_shapes=[
            pltpu.VMEM((n_pad, d_pad), jnp.float32),
            pltpu.VMEM((depth * tk, n_pad), jnp.float32),
            pltpu.VMEM((n_pad, d_pad), jnp.bfloat16),
            pltpu.VMEM((2 * tk, o_pad), jnp.float32),
            pltpu.SemaphoreType.DMA((depth,)),
            pltpu.SemaphoreType.DMA((2,)),
            pltpu.SemaphoreType.DMA,
        ],
        compiler_params=pltpu.CompilerParams(
            vmem_limit_bytes=58 * 1024 * 1024),
    )(feat_p, alpha_p, wqT_p, bq_p, w1T_p, w2T_p, bw_p)

    return out_p[:n, :out_dim]
